# baseline (speedup 1.0000x reference)
import functools

import jax
import jax.numpy as jnp
from jax.experimental import pallas as pl
from jax.experimental.pallas import tpu as pltpu

_STRIDES = (1, 2, 1, 2, 1, 2, 1, 1, 1, 1, 1, 2, 1)
_L = 128


def _cb(c):
    return (c + _L - 1) // _L


def _dw_s1_chunk(slab, ch, w_dim, wtap, dwb):
    bs = []
    for dx in range(3):
        b = None
        for dy in range(3):
            t = slab[dy:dy + ch] * wtap[3 * dy + dx].reshape(1, 1, _L)
            b = t if b is None else b + t
        bs.append(b)
    acc = bs[1]
    z = jnp.zeros((ch, 1, _L), jnp.float32)
    acc = acc + jnp.concatenate([z, bs[0][:, :w_dim - 1, :]], axis=1)
    acc = acc + jnp.concatenate([bs[2][:, 1:, :], z], axis=1)
    return jnp.clip(acc + dwb.reshape(1, 1, _L), 0.0, 6.0)


def _net_kernel(*refs, n_blocks, chans, h0):
    x2d = refs[0]
    c1w, c1s, c1b = refs[1], refs[2], refs[3]
    blk = [refs[4 + 5 * i: 4 + 5 * (i + 1)] for i in range(n_blocks)]
    bw, lns, lnb = refs[4 + 5 * n_blocks: 7 + 5 * n_blocks]
    out_ref = refs[7 + 5 * n_blocks]
    scr = refs[8 + 5 * n_blocks:]
    acts = scr[:n_blocks]
    xcat = scr[n_blocks]
    s2scr = scr[n_blocks + 1:]
    s2i = 0

    h, w_dim = h0, h0
    c0 = chans[0]
    kc = xcat.shape[2]
    s1v = c1s[...]
    b1v = c1b[...]
    xa = x2d[:, 0:h0 + 1, :]
    xb = x2d[:, 1:h0 + 2, :]
    xcat[...] = jnp.concatenate([xa, xb], axis=2)
    cchunk = min(h, max(1, 2048 // w_dim))
    pad0 = _cb(c0) * _L - c0
    for r0 in range(0, h, cchunk):
        ch = min(cchunk, h - r0)
        y = None
        for a in range(2):
            lhs = xcat[r0 + a:r0 + a + ch, 0:w_dim, :]
            lhs = lhs.reshape(ch * w_dim, kc)
            t = jnp.dot(lhs, c1w[a * kc:(a + 1) * kc],
                        preferred_element_type=jnp.float32)
            y = t if y is None else y + t
        y = jnp.clip(y * s1v + b1v, 0.0, 6.0)
        if pad0:
            y = jnp.pad(y, ((0, 0), (0, pad0)))
        acts[0][0, r0:r0 + ch] = y.reshape(ch, w_dim, _L)

    y_last = None
    for bi in range(n_blocks):
        dww_r, dwb_r, pww_r, pws_r, pwb_r = blk[bi]
        stride = _STRIDES[bi]
        cin, cout = chans[bi], chans[bi + 1]
        nbi, nbo = _cb(cin), _cb(cout)
        src, dst = acts[bi], (acts[bi + 1] if bi + 1 < len(acts) else None)
        dww = dww_r[...]
        pws = pws_r[...]
        pwb = pwb_r[...]
        last = bi == n_blocks - 1
        pad_out = nbo * _L - cout

        def pw_store(lhs_blocks, m_rows, r0, ho_, wo_):
            y = None
            for cb in range(nbi):
                t = jnp.dot(lhs_blocks[cb], pww_r[cb * _L:(cb + 1) * _L, :],
                            preferred_element_type=jnp.float32)
                y = t if y is None else y + t
            y = jnp.clip(y * pws + pwb, 0.0, 6.0)
            if last:
                return y
            if pad_out:
                y = jnp.pad(y, ((0, 0), (0, pad_out)))
            for cb in range(nbo):
                dst[cb, r0:r0 + (m_rows // wo_)] = (
                    y[:, cb * _L:(cb + 1) * _L].reshape(
                        m_rows // wo_, wo_, _L))
            return None

        if stride == 1:
            ho, wo = h, w_dim
            chunk = min(h, max(1, 2048 // w_dim),
                        max(1, 131072 // (w_dim * nbi * _L)))
            if last:
                chunk = h
            for r0 in range(0, h, chunk):
                ch = min(chunk, h - r0)
                lo, hi = max(r0 - 1, 0), min(r0 + ch + 1, h)
                zrow = jnp.zeros((1, w_dim, _L), jnp.float32)
                lhs_blocks = []
                for cb in range(nbi):
                    slab = src[cb, lo:hi]
                    if r0 == 0:
                        slab = jnp.concatenate([zrow, slab], axis=0)
                    if r0 + ch == h:
                        slab = jnp.concatenate([slab, zrow], axis=0)
                    wtap = [dww[cb, k] for k in range(9)]
                    dwv = _dw_s1_chunk(slab, ch, w_dim, wtap, dwb_r[cb, 0])
                    lhs_blocks.append(
                        dwv.astype(jnp.bfloat16).reshape(ch * w_dim, _L))
                y_last = pw_store(lhs_blocks, ch * w_dim, r0, ho, wo)
        else:
            ho, wo = h // 2, w_dim // 2
            dwf = s2scr[s2i]
            s2i += 1
            lhs_blocks = []
            for cb in range(nbi):
                bias = dwb_r[cb, 0].reshape(1, _L)
                zc = jnp.zeros((1, _L), jnp.float32)
                for i in range(ho):
                    acc = None
                    for dy in range(3):
                        r = 2 * i + dy - 1
                        if r < 0 or r >= h:
                            continue
                        for dx in range(3):
                            wv = dww[cb, 3 * dy + dx]
                            if dx == 0:
                                t = src[cb, r, pl.ds(1, wo - 1, 2), :] * wv
                                t = jnp.concatenate([zc, t], axis=0)
                            else:
                                t = src[cb, r, pl.ds(dx - 1, wo, 2), :] * wv
                            acc = t if acc is None else acc + t
                    row = jnp.clip(acc + bias, 0.0, 6.0)
                    dwf[i] = row.astype(jnp.bfloat16)
                lhs_blocks.append(dwf[...].reshape(ho * wo, _L))
            y_last = pw_store(lhs_blocks, ho * wo, 0, ho, wo)
        h, w_dim = ho, wo

    feat = jnp.mean(y_last, axis=0, keepdims=True)
    emb = jnp.dot(feat.astype(jnp.bfloat16), bw[...],
                  preferred_element_type=jnp.float32)
    emb = emb * lns[...] + lnb[...]
    ss = jnp.maximum(jnp.sum(emb * emb, axis=1, keepdims=True), 1e-24)
    inv = jax.lax.rsqrt(ss)
    inv = inv * (1.5 - 0.5 * ss * inv * inv)
    out_ref[...] = emb * inv


def kernel(x, conv1_w, conv1_s, conv1_b, b0_dww, b0_dws, b0_dwb, b0_pww, b0_pws, b0_pwb, b1_dww, b1_dws, b1_dwb, b1_pww, b1_pws, b1_pwb, b2_dww, b2_dws, b2_dwb, b2_pww, b2_pws, b2_pwb, b3_dww, b3_dws, b3_dwb, b3_pww, b3_pws, b3_pwb, b4_dww, b4_dws, b4_dwb, b4_pww, b4_pws, b4_pwb, b5_dww, b5_dws, b5_dwb, b5_pww, b5_pws, b5_pwb, b6_dww, b6_dws, b6_dwb, b6_pww, b6_pws, b6_pwb, b7_dww, b7_dws, b7_dwb, b7_pww, b7_pws, b7_pwb, b8_dww, b8_dws, b8_dwb, b8_pww, b8_pws, b8_pwb, b9_dww, b9_dws, b9_dwb, b9_pww, b9_pws, b9_pwb, b10_dww, b10_dws, b10_dwb, b10_pww, b10_pws, b10_pwb, b11_dww, b11_dws, b11_dwb, b11_pww, b11_pws, b11_pwb, b12_dww, b12_dws, b12_dwb, b12_pww, b12_pws, b12_pwb, bottleneck_w, last_bn_s, last_bn_b):
    blocks = [
        (b0_dww, b0_dws, b0_dwb, b0_pww, b0_pws, b0_pwb),
        (b1_dww, b1_dws, b1_dwb, b1_pww, b1_pws, b1_pwb),
        (b2_dww, b2_dws, b2_dwb, b2_pww, b2_pws, b2_pwb),
        (b3_dww, b3_dws, b3_dwb, b3_pww, b3_pws, b3_pwb),
        (b4_dww, b4_dws, b4_dwb, b4_pww, b4_pws, b4_pwb),
        (b5_dww, b5_dws, b5_dwb, b5_pww, b5_pws, b5_pwb),
        (b6_dww, b6_dws, b6_dwb, b6_pww, b6_pws, b6_pwb),
        (b7_dww, b7_dws, b7_dwb, b7_pww, b7_pws, b7_pwb),
        (b8_dww, b8_dws, b8_dwb, b8_pww, b8_pws, b8_pwb),
        (b9_dww, b9_dws, b9_dwb, b9_pww, b9_pws, b9_pwb),
        (b10_dww, b10_dws, b10_dwb, b10_pww, b10_pws, b10_pwb),
        (b11_dww, b11_dws, b11_dwb, b11_pww, b11_pws, b11_pwb),
        (b12_dww, b12_dws, b12_dwb, b12_pww, b12_pws, b12_pwb),
    ]
    n_blocks = len(blocks)
    n, _, hi, wi = x.shape
    h0 = hi // 2
    chans = [blocks[0][0].shape[2]] + [b[3].shape[1] for b in blocks]
    emb_dim = bottleneck_w.shape[1]

    cin0 = x.shape[1]
    s2d = x.astype(jnp.float32).reshape(n, cin0, h0, 2, 2 * h0)
    s2d = jnp.transpose(s2d, (0, 2, 4, 3, 1))
    s2d = s2d.reshape(n, h0, h0, 2, 2, cin0)
    s2d = s2d.reshape(n, h0, h0, 4 * cin0)
    s2dp = jnp.pad(s2d, ((0, 0), (1, 1), (1, 1), (0, 0)))
    x2d = s2dp.astype(jnp.bfloat16)
    wp4 = jnp.pad(conv1_w, ((1, 0), (1, 0), (0, 0), (0, 0)))
    w6 = wp4.reshape(2, 2, 2, 2, cin0, chans[0])
    w6 = jnp.transpose(w6, (0, 2, 3, 1, 4, 5))
    c1w = w6.reshape(16 * cin0, chans[0]).astype(jnp.bfloat16)
    c1s = conv1_s.reshape(1, chans[0])
    c1b = conv1_b.reshape(1, chans[0])

    args = [x2d, c1w, c1s, c1b]
    in_specs = [
        pl.BlockSpec((None, h0 + 2, h0 + 2, 4 * cin0),
                     lambda nn: (nn, 0, 0, 0)),
        pl.BlockSpec(c1w.shape, lambda nn: (0, 0)),
        pl.BlockSpec(c1s.shape, lambda nn: (0, 0)),
        pl.BlockSpec(c1b.shape, lambda nn: (0, 0)),
    ]

    def _cmap(ndim):
        return lambda nn: (0,) * ndim

    for bi, (dww, dws, dwb, pww, pws, pwb) in enumerate(blocks):
        cin, cout = chans[bi], chans[bi + 1]
        nbi = _cb(cin)
        cpi = nbi * _L
        dwsc = dww.reshape(9, cin).astype(jnp.float32) * dws.reshape(1, cin)
        dww4 = jnp.pad(dwsc, ((0, 0), (0, cpi - cin)))
        dww4 = jnp.transpose(dww4.reshape(9, nbi, _L),
                             (1, 0, 2)).reshape(nbi, 9, 1, _L)
        dwb4 = jnp.pad(dwb.reshape(1, cin),
                       ((0, 0), (0, cpi - cin))).reshape(nbi, 1, _L)
        pwwp = jnp.pad(pww, ((0, cpi - cin), (0, 0))).astype(jnp.bfloat16)
        barrs = [dww4, dwb4, pwwp,
                 pws.reshape(1, cout), pwb.reshape(1, cout)]
        args.extend(barrs)
        in_specs.extend(pl.BlockSpec(a.shape, _cmap(a.ndim)) for a in barrs)
    bwb = bottleneck_w.astype(jnp.bfloat16)
    lns = last_bn_s.reshape(1, emb_dim)
    lnb = last_bn_b.reshape(1, emb_dim)
    args.extend([bwb, lns, lnb])
    in_specs.extend([
        pl.BlockSpec(bwb.shape, lambda nn: (0, 0)),
        pl.BlockSpec(lns.shape, lambda nn: (0, 0)),
        pl.BlockSpec(lnb.shape, lambda nn: (0, 0)),
    ])

    scratch = []
    hh = h0
    scratch.append(pltpu.VMEM((_cb(chans[0]), hh, hh, _L), jnp.float32))
    for bi in range(n_blocks - 1):
        if _STRIDES[bi] == 2:
            hh //= 2
        scratch.append(pltpu.VMEM((_cb(chans[bi + 1]), hh, hh, _L),
                                  jnp.float32))
    scratch.append(pltpu.VMEM((h0 + 2, h0 + 1, 8 * cin0), jnp.bfloat16))
    hh = h0
    for bi in range(n_blocks):
        if _STRIDES[bi] == 2:
            hh //= 2
            scratch.append(pltpu.VMEM((hh, hh, _L), jnp.bfloat16))

    fn = functools.partial(_net_kernel, n_blocks=n_blocks, chans=chans,
                           h0=h0)
    out = pl.pallas_call(
        fn,
        out_shape=jax.ShapeDtypeStruct((n, 1, emb_dim), jnp.float32),
        grid=(n,),
        in_specs=in_specs,
        out_specs=pl.BlockSpec((None, 1, emb_dim), lambda nn: (nn, 0, 0)),
        scratch_shapes=scratch,
        compiler_params=pltpu.CompilerParams(
            dimension_semantics=("parallel",),
            vmem_limit_bytes=48 << 20,
        ),
    )(*args)
    return out.reshape(n, emb_dim)


# speedup vs baseline: 1.0017x; 1.0017x over previous
import functools

import jax
import jax.numpy as jnp
from jax.experimental import pallas as pl
from jax.experimental.pallas import tpu as pltpu

_STRIDES = (1, 2, 1, 2, 1, 2, 1, 1, 1, 1, 1, 2, 1)
_L = 128


def _cb(c):
    return (c + _L - 1) // _L


def _dw_s1_chunk(slab, ch, w_dim, wtap, dwb):
    bs = []
    for dx in range(3):
        b = None
        for dy in range(3):
            t = slab[dy:dy + ch] * wtap[3 * dy + dx].reshape(1, 1, _L)
            b = t if b is None else b + t
        bs.append(b)
    acc = bs[1]
    z = jnp.zeros((ch, 1, _L), jnp.float32)
    acc = acc + jnp.concatenate([z, bs[0][:, :w_dim - 1, :]], axis=1)
    acc = acc + jnp.concatenate([bs[2][:, 1:, :], z], axis=1)
    return jnp.clip(acc + dwb.reshape(1, 1, _L), 0.0, 6.0)


def _net_kernel(*refs, n_blocks, chans, h0):
    x2d = refs[0]
    c1w, c1s, c1b = refs[1], refs[2], refs[3]
    blk = [refs[4 + 5 * i: 4 + 5 * (i + 1)] for i in range(n_blocks)]
    bw, lns, lnb = refs[4 + 5 * n_blocks: 7 + 5 * n_blocks]
    out_ref = refs[7 + 5 * n_blocks]
    scr = refs[8 + 5 * n_blocks:]
    acts = scr[:n_blocks]
    xcat = scr[n_blocks]
    s2scr = scr[n_blocks + 1:]
    s2i = 0

    h, w_dim = h0, h0
    c0 = chans[0]
    kc = xcat.shape[2]
    s1v = c1s[...]
    b1v = c1b[...]
    xa = x2d[:, 0:h0 + 1, :]
    xb = x2d[:, 1:h0 + 2, :]
    xcat[...] = jnp.concatenate([xa, xb], axis=2)
    cchunk = min(h, max(1, 2048 // w_dim))
    pad0 = _cb(c0) * _L - c0
    for r0 in range(0, h, cchunk):
        ch = min(cchunk, h - r0)
        y = None
        for a in range(2):
            lhs = xcat[r0 + a:r0 + a + ch, 0:w_dim, :]
            lhs = lhs.reshape(ch * w_dim, kc)
            t = jnp.dot(lhs, c1w[a * kc:(a + 1) * kc],
                        preferred_element_type=jnp.float32)
            y = t if y is None else y + t
        y = jnp.clip(y * s1v + b1v, 0.0, 6.0)
        if pad0:
            y = jnp.pad(y, ((0, 0), (0, pad0)))
        acts[0][0, r0:r0 + ch] = y.reshape(ch, w_dim, _L)

    y_last = None
    for bi in range(n_blocks):
        dww_r, dwb_r, pww_r, pws_r, pwb_r = blk[bi]
        stride = _STRIDES[bi]
        cin, cout = chans[bi], chans[bi + 1]
        nbi, nbo = _cb(cin), _cb(cout)
        src, dst = acts[bi], (acts[bi + 1] if bi + 1 < len(acts) else None)
        dww = dww_r[...]
        pws = pws_r[...]
        pwb = pwb_r[...]
        last = bi == n_blocks - 1
        pad_out = nbo * _L - cout

        def pw_store(lhs_blocks, m_rows, r0, ho_, wo_):
            y = None
            for cb in range(nbi):
                t = jnp.dot(lhs_blocks[cb], pww_r[cb * _L:(cb + 1) * _L, :],
                            preferred_element_type=jnp.float32)
                y = t if y is None else y + t
            y = jnp.clip(y * pws + pwb, 0.0, 6.0)
            if last:
                return y
            if pad_out:
                y = jnp.pad(y, ((0, 0), (0, pad_out)))
            for cb in range(nbo):
                dst[cb, r0:r0 + (m_rows // wo_)] = (
                    y[:, cb * _L:(cb + 1) * _L].reshape(
                        m_rows // wo_, wo_, _L))
            return None

        if stride == 1:
            ho, wo = h, w_dim
            chunk = min(h, max(1, 2048 // w_dim),
                        max(1, 131072 // (w_dim * nbi * _L)))
            if last:
                chunk = h
            for r0 in range(0, h, chunk):
                ch = min(chunk, h - r0)
                lo, hi = max(r0 - 1, 0), min(r0 + ch + 1, h)
                zrow = jnp.zeros((1, w_dim, _L), jnp.float32)
                lhs_blocks = []
                for cb in range(nbi):
                    slab = src[cb, lo:hi]
                    if r0 == 0:
                        slab = jnp.concatenate([zrow, slab], axis=0)
                    if r0 + ch == h:
                        slab = jnp.concatenate([slab, zrow], axis=0)
                    wtap = [dww[cb, k] for k in range(9)]
                    dwv = _dw_s1_chunk(slab, ch, w_dim, wtap, dwb_r[cb, 0])
                    lhs_blocks.append(
                        dwv.astype(jnp.bfloat16).reshape(ch * w_dim, _L))
                y_last = pw_store(lhs_blocks, ch * w_dim, r0, ho, wo)
        else:
            ho, wo = h // 2, w_dim // 2
            dwf = s2scr[s2i]
            s2i += 1
            lhs_blocks = []
            for cb in range(nbi):
                bias = dwb_r[cb, 0].reshape(1, _L)
                zc = jnp.zeros((1, _L), jnp.float32)
                for i in range(ho):
                    acc = None
                    for dy in range(3):
                        r = 2 * i + dy - 1
                        if r < 0 or r >= h:
                            continue
                        for dx in range(3):
                            wv = dww[cb, 3 * dy + dx]
                            if dx == 0:
                                t = src[cb, r, pl.ds(1, wo - 1, 2), :] * wv
                                t = jnp.concatenate([zc, t], axis=0)
                            else:
                                t = src[cb, r, pl.ds(dx - 1, wo, 2), :] * wv
                            acc = t if acc is None else acc + t
                    row = jnp.clip(acc + bias, 0.0, 6.0)
                    dwf[i] = row.astype(jnp.bfloat16)
                lhs_blocks.append(dwf[...].reshape(ho * wo, _L))
            y_last = pw_store(lhs_blocks, ho * wo, 0, ho, wo)
        h, w_dim = ho, wo

    feat = jnp.mean(y_last, axis=0, keepdims=True)
    emb = jnp.dot(feat.astype(jnp.bfloat16), bw[...],
                  preferred_element_type=jnp.float32)
    emb = emb * lns[...] + lnb[...]
    ss = jnp.maximum(jnp.sum(emb * emb, axis=1, keepdims=True), 1e-24)
    inv = jax.lax.rsqrt(ss)
    inv = inv * (1.5 - 0.5 * ss * inv * inv)
    out_ref[...] = emb * inv


def kernel(x, conv1_w, conv1_s, conv1_b, b0_dww, b0_dws, b0_dwb, b0_pww, b0_pws, b0_pwb, b1_dww, b1_dws, b1_dwb, b1_pww, b1_pws, b1_pwb, b2_dww, b2_dws, b2_dwb, b2_pww, b2_pws, b2_pwb, b3_dww, b3_dws, b3_dwb, b3_pww, b3_pws, b3_pwb, b4_dww, b4_dws, b4_dwb, b4_pww, b4_pws, b4_pwb, b5_dww, b5_dws, b5_dwb, b5_pww, b5_pws, b5_pwb, b6_dww, b6_dws, b6_dwb, b6_pww, b6_pws, b6_pwb, b7_dww, b7_dws, b7_dwb, b7_pww, b7_pws, b7_pwb, b8_dww, b8_dws, b8_dwb, b8_pww, b8_pws, b8_pwb, b9_dww, b9_dws, b9_dwb, b9_pww, b9_pws, b9_pwb, b10_dww, b10_dws, b10_dwb, b10_pww, b10_pws, b10_pwb, b11_dww, b11_dws, b11_dwb, b11_pww, b11_pws, b11_pwb, b12_dww, b12_dws, b12_dwb, b12_pww, b12_pws, b12_pwb, bottleneck_w, last_bn_s, last_bn_b):
    blocks = [
        (b0_dww, b0_dws, b0_dwb, b0_pww, b0_pws, b0_pwb),
        (b1_dww, b1_dws, b1_dwb, b1_pww, b1_pws, b1_pwb),
        (b2_dww, b2_dws, b2_dwb, b2_pww, b2_pws, b2_pwb),
        (b3_dww, b3_dws, b3_dwb, b3_pww, b3_pws, b3_pwb),
        (b4_dww, b4_dws, b4_dwb, b4_pww, b4_pws, b4_pwb),
        (b5_dww, b5_dws, b5_dwb, b5_pww, b5_pws, b5_pwb),
        (b6_dww, b6_dws, b6_dwb, b6_pww, b6_pws, b6_pwb),
        (b7_dww, b7_dws, b7_dwb, b7_pww, b7_pws, b7_pwb),
        (b8_dww, b8_dws, b8_dwb, b8_pww, b8_pws, b8_pwb),
        (b9_dww, b9_dws, b9_dwb, b9_pww, b9_pws, b9_pwb),
        (b10_dww, b10_dws, b10_dwb, b10_pww, b10_pws, b10_pwb),
        (b11_dww, b11_dws, b11_dwb, b11_pww, b11_pws, b11_pwb),
        (b12_dww, b12_dws, b12_dwb, b12_pww, b12_pws, b12_pwb),
    ]
    n_blocks = len(blocks)
    n, _, hi, wi = x.shape
    h0 = hi // 2
    chans = [blocks[0][0].shape[2]] + [b[3].shape[1] for b in blocks]
    emb_dim = bottleneck_w.shape[1]

    cin0 = x.shape[1]
    s2d = x.astype(jnp.float32).reshape(n, cin0, h0, 2, 2 * h0)
    s2d = jnp.transpose(s2d, (0, 2, 4, 3, 1))
    s2d = s2d.reshape(n, h0, h0, 2, 2, cin0)
    s2d = s2d.reshape(n, h0, h0, 4 * cin0)
    s2dp = jnp.pad(s2d, ((0, 0), (1, 1), (1, 1), (0, 0)))
    x2d = s2dp.astype(jnp.bfloat16)
    wp4 = jnp.pad(conv1_w, ((1, 0), (1, 0), (0, 0), (0, 0)))
    w6 = wp4.reshape(2, 2, 2, 2, cin0, chans[0])
    w6 = jnp.transpose(w6, (0, 2, 3, 1, 4, 5))
    c1w = w6.reshape(16 * cin0, chans[0]).astype(jnp.bfloat16)
    c1s = conv1_s.reshape(1, chans[0])
    c1b = conv1_b.reshape(1, chans[0])

    args = [x2d, c1w, c1s, c1b]
    in_specs = [
        pl.BlockSpec((None, h0 + 2, h0 + 2, 4 * cin0),
                     lambda nn: (nn, 0, 0, 0)),
        pl.BlockSpec(c1w.shape, lambda nn: (0, 0)),
        pl.BlockSpec(c1s.shape, lambda nn: (0, 0)),
        pl.BlockSpec(c1b.shape, lambda nn: (0, 0)),
    ]

    def _cmap(ndim):
        return lambda nn: (0,) * ndim

    for bi, (dww, dws, dwb, pww, pws, pwb) in enumerate(blocks):
        cin, cout = chans[bi], chans[bi + 1]
        nbi = _cb(cin)
        cpi = nbi * _L
        dwsc = dww.reshape(9, cin).astype(jnp.float32) * dws.reshape(1, cin)
        dww4 = jnp.pad(dwsc, ((0, 0), (0, cpi - cin)))
        dww4 = jnp.transpose(dww4.reshape(9, nbi, _L),
                             (1, 0, 2)).reshape(nbi, 9, 1, _L)
        dwb4 = jnp.pad(dwb.reshape(1, cin),
                       ((0, 0), (0, cpi - cin))).reshape(nbi, 1, _L)
        pwwp = jnp.pad(pww, ((0, cpi - cin), (0, 0))).astype(jnp.bfloat16)
        barrs = [dww4, dwb4, pwwp,
                 pws.reshape(1, cout), pwb.reshape(1, cout)]
        args.extend(barrs)
        in_specs.extend(pl.BlockSpec(a.shape, _cmap(a.ndim)) for a in barrs)
    bwb = bottleneck_w.astype(jnp.bfloat16)
    lns = last_bn_s.reshape(1, emb_dim)
    lnb = last_bn_b.reshape(1, emb_dim)
    args.extend([bwb, lns, lnb])
    in_specs.extend([
        pl.BlockSpec(bwb.shape, lambda nn: (0, 0)),
        pl.BlockSpec(lns.shape, lambda nn: (0, 0)),
        pl.BlockSpec(lnb.shape, lambda nn: (0, 0)),
    ])

    scratch = []
    hh = h0
    scratch.append(pltpu.VMEM((_cb(chans[0]), hh, hh, _L), jnp.float32))
    for bi in range(n_blocks - 1):
        if _STRIDES[bi] == 2:
            hh //= 2
        scratch.append(pltpu.VMEM((_cb(chans[bi + 1]), hh, hh, _L),
                                  jnp.float32))
    scratch.append(pltpu.VMEM((h0 + 2, h0 + 1, 8 * cin0), jnp.bfloat16))
    hh = h0
    for bi in range(n_blocks):
        if _STRIDES[bi] == 2:
            hh //= 2
            scratch.append(pltpu.VMEM((hh, hh, _L), jnp.bfloat16))

    fn = functools.partial(_net_kernel, n_blocks=n_blocks, chans=chans,
                           h0=h0)
    out = pl.pallas_call(
        fn,
        out_shape=jax.ShapeDtypeStruct((n, 1, emb_dim), jnp.float32),
        grid=(n,),
        in_specs=in_specs,
        out_specs=pl.BlockSpec((None, 1, emb_dim), lambda nn: (nn, 0, 0)),
        scratch_shapes=scratch,
        compiler_params=pltpu.CompilerParams(
            dimension_semantics=("arbitrary",),
            vmem_limit_bytes=48 << 20,
        ),
    )(*args)
    return out.reshape(n, emb_dim)


# speedup vs baseline: 1.0063x; 1.0046x over previous
import functools

import jax
import jax.numpy as jnp
from jax.experimental import pallas as pl
from jax.experimental.pallas import tpu as pltpu

_STRIDES = (1, 2, 1, 2, 1, 2, 1, 1, 1, 1, 1, 2, 1)
_L = 128


def _cb(c):
    return (c + _L - 1) // _L


def _dw_s1_chunk(slab, ch, w_dim, wtap, dwb):
    bs = []
    for dx in range(3):
        b = None
        for dy in range(3):
            t = slab[dy:dy + ch] * wtap[3 * dy + dx].reshape(1, 1, _L)
            b = t if b is None else b + t
        bs.append(b)
    acc = bs[1]
    z = jnp.zeros((ch, 1, _L), jnp.float32)
    acc = acc + jnp.concatenate([z, bs[0][:, :w_dim - 1, :]], axis=1)
    acc = acc + jnp.concatenate([bs[2][:, 1:, :], z], axis=1)
    return jnp.clip(acc + dwb.reshape(1, 1, _L), 0.0, 6.0)


def _net_kernel(*refs, n_blocks, chans, h0, g_imgs):
    x2d_all = refs[0]
    c1w, c1s, c1b = refs[1], refs[2], refs[3]
    blk = [refs[4 + 5 * i: 4 + 5 * (i + 1)] for i in range(n_blocks)]
    bw, lns, lnb = refs[4 + 5 * n_blocks: 7 + 5 * n_blocks]
    out_all = refs[7 + 5 * n_blocks]
    scr = refs[8 + 5 * n_blocks:]
    per_img = len(scr) // g_imgs
    for s_img in range(g_imgs):
        _net_one(x2d_all.at[s_img], out_all.at[s_img],
                 scr[s_img * per_img:(s_img + 1) * per_img],
                 c1w, c1s, c1b, blk, bw, lns, lnb,
                 n_blocks=n_blocks, chans=chans, h0=h0)


def _net_one(x2d, out_ref, scr, c1w, c1s, c1b, blk, bw, lns, lnb,
             *, n_blocks, chans, h0):
    acts = scr[:n_blocks]
    xcat = scr[n_blocks]
    s2scr = scr[n_blocks + 1:]
    s2i = 0

    h, w_dim = h0, h0
    c0 = chans[0]
    kc = xcat.shape[2]
    s1v = c1s[...]
    b1v = c1b[...]
    xa = x2d[:, 0:h0 + 1, :]
    xb = x2d[:, 1:h0 + 2, :]
    xcat[...] = jnp.concatenate([xa, xb], axis=2)
    cchunk = min(h, max(1, 2048 // w_dim))
    pad0 = _cb(c0) * _L - c0
    for r0 in range(0, h, cchunk):
        ch = min(cchunk, h - r0)
        y = None
        for a in range(2):
            lhs = xcat[r0 + a:r0 + a + ch, 0:w_dim, :]
            lhs = lhs.reshape(ch * w_dim, kc)
            t = jnp.dot(lhs, c1w[a * kc:(a + 1) * kc],
                        preferred_element_type=jnp.float32)
            y = t if y is None else y + t
        y = jnp.clip(y * s1v + b1v, 0.0, 6.0)
        if pad0:
            y = jnp.pad(y, ((0, 0), (0, pad0)))
        acts[0][0, r0:r0 + ch] = y.reshape(ch, w_dim, _L)

    y_last = None
    for bi in range(n_blocks):
        dww_r, dwb_r, pww_r, pws_r, pwb_r = blk[bi]
        stride = _STRIDES[bi]
        cin, cout = chans[bi], chans[bi + 1]
        nbi, nbo = _cb(cin), _cb(cout)
        src, dst = acts[bi], (acts[bi + 1] if bi + 1 < len(acts) else None)
        dww = dww_r[...]
        pws = pws_r[...]
        pwb = pwb_r[...]
        last = bi == n_blocks - 1
        pad_out = nbo * _L - cout

        def pw_store(lhs_blocks, m_rows, r0, ho_, wo_):
            y = None
            for cb in range(nbi):
                t = jnp.dot(lhs_blocks[cb], pww_r[cb * _L:(cb + 1) * _L, :],
                            preferred_element_type=jnp.float32)
                y = t if y is None else y + t
            y = jnp.clip(y * pws + pwb, 0.0, 6.0)
            if last:
                return y
            if pad_out:
                y = jnp.pad(y, ((0, 0), (0, pad_out)))
            for cb in range(nbo):
                dst[cb, r0:r0 + (m_rows // wo_)] = (
                    y[:, cb * _L:(cb + 1) * _L].reshape(
                        m_rows // wo_, wo_, _L))
            return None

        if stride == 1:
            ho, wo = h, w_dim
            chunk = min(h, max(1, 2048 // w_dim),
                        max(1, 131072 // (w_dim * nbi * _L)))
            if last:
                chunk = h
            for r0 in range(0, h, chunk):
                ch = min(chunk, h - r0)
                lo, hi = max(r0 - 1, 0), min(r0 + ch + 1, h)
                zrow = jnp.zeros((1, w_dim, _L), jnp.float32)
                lhs_blocks = []
                for cb in range(nbi):
                    slab = src[cb, lo:hi]
                    if r0 == 0:
                        slab = jnp.concatenate([zrow, slab], axis=0)
                    if r0 + ch == h:
                        slab = jnp.concatenate([slab, zrow], axis=0)
                    wtap = [dww[cb, k] for k in range(9)]
                    dwv = _dw_s1_chunk(slab, ch, w_dim, wtap, dwb_r[cb, 0])
                    lhs_blocks.append(
                        dwv.astype(jnp.bfloat16).reshape(ch * w_dim, _L))
                y_last = pw_store(lhs_blocks, ch * w_dim, r0, ho, wo)
        else:
            ho, wo = h // 2, w_dim // 2
            dwf = s2scr[s2i]
            s2i += 1
            lhs_blocks = []
            for cb in range(nbi):
                bias = dwb_r[cb, 0].reshape(1, _L)
                zc = jnp.zeros((1, _L), jnp.float32)
                for i in range(ho):
                    acc = None
                    for dy in range(3):
                        r = 2 * i + dy - 1
                        if r < 0 or r >= h:
                            continue
                        for dx in range(3):
                            wv = dww[cb, 3 * dy + dx]
                            if dx == 0:
                                t = src[cb, r, pl.ds(1, wo - 1, 2), :] * wv
                                t = jnp.concatenate([zc, t], axis=0)
                            else:
                                t = src[cb, r, pl.ds(dx - 1, wo, 2), :] * wv
                            acc = t if acc is None else acc + t
                    row = jnp.clip(acc + bias, 0.0, 6.0)
                    dwf[i] = row.astype(jnp.bfloat16)
                lhs_blocks.append(dwf[...].reshape(ho * wo, _L))
            y_last = pw_store(lhs_blocks, ho * wo, 0, ho, wo)
        h, w_dim = ho, wo

    feat = jnp.mean(y_last, axis=0, keepdims=True)
    emb = jnp.dot(feat.astype(jnp.bfloat16), bw[...],
                  preferred_element_type=jnp.float32)
    emb = emb * lns[...] + lnb[...]
    ss = jnp.maximum(jnp.sum(emb * emb, axis=1, keepdims=True), 1e-24)
    inv = jax.lax.rsqrt(ss)
    inv = inv * (1.5 - 0.5 * ss * inv * inv)
    out_ref[...] = emb * inv


def kernel(x, conv1_w, conv1_s, conv1_b, b0_dww, b0_dws, b0_dwb, b0_pww, b0_pws, b0_pwb, b1_dww, b1_dws, b1_dwb, b1_pww, b1_pws, b1_pwb, b2_dww, b2_dws, b2_dwb, b2_pww, b2_pws, b2_pwb, b3_dww, b3_dws, b3_dwb, b3_pww, b3_pws, b3_pwb, b4_dww, b4_dws, b4_dwb, b4_pww, b4_pws, b4_pwb, b5_dww, b5_dws, b5_dwb, b5_pww, b5_pws, b5_pwb, b6_dww, b6_dws, b6_dwb, b6_pww, b6_pws, b6_pwb, b7_dww, b7_dws, b7_dwb, b7_pww, b7_pws, b7_pwb, b8_dww, b8_dws, b8_dwb, b8_pww, b8_pws, b8_pwb, b9_dww, b9_dws, b9_dwb, b9_pww, b9_pws, b9_pwb, b10_dww, b10_dws, b10_dwb, b10_pww, b10_pws, b10_pwb, b11_dww, b11_dws, b11_dwb, b11_pww, b11_pws, b11_pwb, b12_dww, b12_dws, b12_dwb, b12_pww, b12_pws, b12_pwb, bottleneck_w, last_bn_s, last_bn_b):
    blocks = [
        (b0_dww, b0_dws, b0_dwb, b0_pww, b0_pws, b0_pwb),
        (b1_dww, b1_dws, b1_dwb, b1_pww, b1_pws, b1_pwb),
        (b2_dww, b2_dws, b2_dwb, b2_pww, b2_pws, b2_pwb),
        (b3_dww, b3_dws, b3_dwb, b3_pww, b3_pws, b3_pwb),
        (b4_dww, b4_dws, b4_dwb, b4_pww, b4_pws, b4_pwb),
        (b5_dww, b5_dws, b5_dwb, b5_pww, b5_pws, b5_pwb),
        (b6_dww, b6_dws, b6_dwb, b6_pww, b6_pws, b6_pwb),
        (b7_dww, b7_dws, b7_dwb, b7_pww, b7_pws, b7_pwb),
        (b8_dww, b8_dws, b8_dwb, b8_pww, b8_pws, b8_pwb),
        (b9_dww, b9_dws, b9_dwb, b9_pww, b9_pws, b9_pwb),
        (b10_dww, b10_dws, b10_dwb, b10_pww, b10_pws, b10_pwb),
        (b11_dww, b11_dws, b11_dwb, b11_pww, b11_pws, b11_pwb),
        (b12_dww, b12_dws, b12_dwb, b12_pww, b12_pws, b12_pwb),
    ]
    n_blocks = len(blocks)
    n, _, hi, wi = x.shape
    h0 = hi // 2
    chans = [blocks[0][0].shape[2]] + [b[3].shape[1] for b in blocks]
    emb_dim = bottleneck_w.shape[1]

    cin0 = x.shape[1]
    s2d = x.astype(jnp.float32).reshape(n, cin0, h0, 2, 2 * h0)
    s2d = jnp.transpose(s2d, (0, 2, 4, 3, 1))
    s2d = s2d.reshape(n, h0, h0, 2, 2, cin0)
    s2d = s2d.reshape(n, h0, h0, 4 * cin0)
    s2dp = jnp.pad(s2d, ((0, 0), (1, 1), (1, 1), (0, 0)))
    x2d = s2dp.astype(jnp.bfloat16)
    wp4 = jnp.pad(conv1_w, ((1, 0), (1, 0), (0, 0), (0, 0)))
    w6 = wp4.reshape(2, 2, 2, 2, cin0, chans[0])
    w6 = jnp.transpose(w6, (0, 2, 3, 1, 4, 5))
    c1w = w6.reshape(16 * cin0, chans[0]).astype(jnp.bfloat16)
    c1s = conv1_s.reshape(1, chans[0])
    c1b = conv1_b.reshape(1, chans[0])

    g_imgs = 2
    args = [x2d, c1w, c1s, c1b]
    in_specs = [
        pl.BlockSpec((g_imgs, h0 + 2, h0 + 2, 4 * cin0),
                     lambda nn: (nn, 0, 0, 0)),
        pl.BlockSpec(c1w.shape, lambda nn: (0, 0)),
        pl.BlockSpec(c1s.shape, lambda nn: (0, 0)),
        pl.BlockSpec(c1b.shape, lambda nn: (0, 0)),
    ]

    def _cmap(ndim):
        return lambda nn: (0,) * ndim

    for bi, (dww, dws, dwb, pww, pws, pwb) in enumerate(blocks):
        cin, cout = chans[bi], chans[bi + 1]
        nbi = _cb(cin)
        cpi = nbi * _L
        dwsc = dww.reshape(9, cin).astype(jnp.float32) * dws.reshape(1, cin)
        dww4 = jnp.pad(dwsc, ((0, 0), (0, cpi - cin)))
        dww4 = jnp.transpose(dww4.reshape(9, nbi, _L),
                             (1, 0, 2)).reshape(nbi, 9, 1, _L)
        dwb4 = jnp.pad(dwb.reshape(1, cin),
                       ((0, 0), (0, cpi - cin))).reshape(nbi, 1, _L)
        pwwp = jnp.pad(pww, ((0, cpi - cin), (0, 0))).astype(jnp.bfloat16)
        barrs = [dww4, dwb4, pwwp,
                 pws.reshape(1, cout), pwb.reshape(1, cout)]
        args.extend(barrs)
        in_specs.extend(pl.BlockSpec(a.shape, _cmap(a.ndim)) for a in barrs)
    bwb = bottleneck_w.astype(jnp.bfloat16)
    lns = last_bn_s.reshape(1, emb_dim)
    lnb = last_bn_b.reshape(1, emb_dim)
    args.extend([bwb, lns, lnb])
    in_specs.extend([
        pl.BlockSpec(bwb.shape, lambda nn: (0, 0)),
        pl.BlockSpec(lns.shape, lambda nn: (0, 0)),
        pl.BlockSpec(lnb.shape, lambda nn: (0, 0)),
    ])

    scratch = []
    for _ in range(g_imgs):
        hh = h0
        scratch.append(pltpu.VMEM((_cb(chans[0]), hh, hh, _L), jnp.float32))
        for bi in range(n_blocks - 1):
            if _STRIDES[bi] == 2:
                hh //= 2
            scratch.append(pltpu.VMEM((_cb(chans[bi + 1]), hh, hh, _L),
                                      jnp.float32))
        scratch.append(pltpu.VMEM((h0 + 2, h0 + 1, 8 * cin0), jnp.bfloat16))
        hh = h0
        for bi in range(n_blocks):
            if _STRIDES[bi] == 2:
                hh //= 2
                scratch.append(pltpu.VMEM((hh, hh, _L), jnp.bfloat16))

    fn = functools.partial(_net_kernel, n_blocks=n_blocks, chans=chans,
                           h0=h0, g_imgs=g_imgs)
    out = pl.pallas_call(
        fn,
        out_shape=jax.ShapeDtypeStruct((n, 1, emb_dim), jnp.float32),
        grid=(n // g_imgs,),
        in_specs=in_specs,
        out_specs=pl.BlockSpec((g_imgs, 1, emb_dim), lambda nn: (nn, 0, 0)),
        scratch_shapes=scratch,
        compiler_params=pltpu.CompilerParams(
            dimension_semantics=("parallel",),
            vmem_limit_bytes=48 << 20,
        ),
    )(*args)
    return out.reshape(n, emb_dim)
